# baseline (speedup 1.0000x reference)
import functools

import jax
import jax.numpy as jnp
from jax import lax
from jax.experimental import pallas as pl
from jax.experimental.pallas import tpu as pltpu
from jax.experimental.pallas import tpu_sc as plsc

N = 10000
E = 320000
D = 128

NCC = 2
NS = 16
NW = NCC * NS

NPAD = 10112
TRASH = 10000
STRIPE = NPAD // NS

K = 80
NCH = 125
NBUF = 2
EPW = K * NCH

RB = 1264


def _sc_deg_body(dst_hbm, deg_out, didx, deg):
    cid = lax.axis_index("c")
    sid = lax.axis_index("s")
    wid = cid * NS + sid
    pltpu.sync_copy(dst_hbm.at[wid], didx)

    def zero(i, _):
        deg[pl.ds(i * 16, 16)] = jnp.zeros((16,), jnp.float32)
        return _

    lax.fori_loop(0, NPAD // 16, zero, None)

    def acc(i, _):
        idx = didx[pl.ds(i * 16, 16)]
        plsc.addupdate_scatter(deg, [idx], jnp.ones((16,), jnp.float32))
        return _

    lax.fori_loop(0, EPW // 16, acc, None)
    pltpu.sync_copy(deg, deg_out.at[wid])


_sc_deg = pl.kernel(
    _sc_deg_body,
    out_type=jax.ShapeDtypeStruct((NW, NPAD), jnp.float32),
    mesh=plsc.VectorSubcoreMesh(core_axis_name="c", subcore_axis_name="s",
                                num_cores=NCC, num_subcores=NS),
    scratch_types=[
        pltpu.VMEM((EPW,), jnp.int32),
        pltpu.VMEM((NPAD,), jnp.float32),
    ],
    compiler_params=pltpu.CompilerParams(needs_layout_passes=False),
)


def _sc_scatter_body(hs, src1, dst3, zrows, out, sidx, didx, rows,
                     accum, gsem, ssem):
    cid = lax.axis_index("c")
    sid = lax.axis_index("s")
    wid = cid * NS + sid

    pltpu.sync_copy(src1.at[wid], sidx)
    pltpu.sync_copy(dst3.at[wid], didx)

    pltpu.sync_copy(zrows, accum.at[pl.ds(sid * STRIPE, STRIPE)])
    plsc.subcore_barrier()

    for b in range(NBUF):
        pltpu.async_copy(hs.at[sidx.at[pl.ds(b * K, K)]], rows.at[b],
                         gsem.at[b])

    def grp(g, _):
        for b in range(NBUF):
            j = g * NBUF + b
            pltpu.make_async_copy(hs.at[sidx.at[pl.ds(j * K, K)]],
                                  rows.at[b], gsem.at[b]).wait()
            pltpu.async_copy(rows.at[b], accum.at[didx.at[j]], ssem.at[b],
                             add=True)
        for b in range(NBUF):
            j = g * NBUF + b
            pltpu.make_async_copy(rows.at[b], accum.at[didx.at[j]],
                                  ssem.at[b]).wait()
            nj = j + NBUF

            @pl.when(nj < NCH)
            def _issue():
                pltpu.async_copy(hs.at[sidx.at[pl.ds(nj * K, K)]],
                                 rows.at[b], gsem.at[b])

        return _

    lax.fori_loop(0, NCH // NBUF, grp, None)
    for j in range(NCH - NCH % NBUF, NCH):
        b = j % NBUF
        pltpu.make_async_copy(hs.at[sidx.at[pl.ds(j * K, K)]],
                              rows.at[b], gsem.at[b]).wait()
        pltpu.async_copy(rows.at[b], accum.at[didx.at[j]], ssem.at[b],
                         add=True)
        pltpu.make_async_copy(rows.at[b], accum.at[didx.at[j]],
                              ssem.at[b]).wait()
    plsc.subcore_barrier()

    pltpu.sync_copy(accum.at[pl.ds(sid * STRIPE, STRIPE)],
                    out.at[cid, pl.ds(sid * STRIPE, STRIPE)])


_sc_scatter = pl.kernel(
    _sc_scatter_body,
    out_type=jax.ShapeDtypeStruct((NCC, NPAD, D), jnp.float32),
    mesh=plsc.VectorSubcoreMesh(core_axis_name="c", subcore_axis_name="s",
                                num_cores=NCC, num_subcores=NS),
    scratch_types=[
        pltpu.VMEM((EPW,), jnp.int32),
        pltpu.VMEM((NCH, K), jnp.int32),
        pltpu.VMEM((NBUF, K, D), jnp.float32),
        pltpu.VMEM_SHARED((NPAD, D), jnp.float32),
        pltpu.SemaphoreType.DMA((NBUF,)),
        pltpu.SemaphoreType.DMA((NBUF,)),
    ],
    compiler_params=pltpu.CompilerParams(needs_layout_passes=False),
)


def _tc_mmdvhs_body(x_ref, degT_ref, w_ref, dv_ref, hs_ref):
    s = jnp.sum(degT_ref[...], axis=1, keepdims=True) + 1.0
    dv = lax.rsqrt(s)
    dv_ref[...] = dv
    hs_ref[...] = jnp.dot(x_ref[...], w_ref[...],
                          preferred_element_type=jnp.float32) * dv


def _tc_fuse1_body(a_ref, b_ref, hs_ref, dv_ref, b1_ref, w2_ref, o_ref):
    z = (a_ref[0] + b_ref[0] + hs_ref[...]) * dv_ref[...] + b1_ref[...]
    z = jnp.maximum(z, 0.0)
    o_ref[...] = jnp.dot(z, w2_ref[...],
                         preferred_element_type=jnp.float32) * dv_ref[...]


def _tc_fuse2_body(a_ref, b_ref, hs_ref, dv_ref, b2_ref, o_ref):
    o_ref[...] = (a_ref[0] + b_ref[0] + hs_ref[...]) * dv_ref[...] + b2_ref[...]


_GRID = NPAD // RB


def _row_spec(w):
    return pl.BlockSpec((RB, w), lambda i: (i, 0))


def _full_spec(h, w):
    return pl.BlockSpec((h, w), lambda i: (0, 0))


def _part_spec(c):
    return pl.BlockSpec((1, RB, D), lambda i, c=c: (c, i, 0))


_tc_mmdvhs = pl.pallas_call(
    _tc_mmdvhs_body,
    grid=(_GRID,),
    in_specs=[_row_spec(D), _row_spec(NW), _full_spec(D, D)],
    out_specs=(_row_spec(1), _row_spec(D)),
    out_shape=(jax.ShapeDtypeStruct((NPAD, 1), jnp.float32),
               jax.ShapeDtypeStruct((NPAD, D), jnp.float32)),
)

_tc_fuse1 = pl.pallas_call(
    _tc_fuse1_body,
    grid=(_GRID,),
    in_specs=[_part_spec(0), _part_spec(1), _row_spec(D), _row_spec(1),
              _full_spec(1, D), _full_spec(D, D)],
    out_specs=_row_spec(D),
    out_shape=jax.ShapeDtypeStruct((NPAD, D), jnp.float32),
)

RB2 = 2000


def _p2_spec(c):
    return pl.BlockSpec((1, RB2, D), lambda i, c=c: (c, i, 0))


_tc_fuse2 = pl.pallas_call(
    _tc_fuse2_body,
    grid=(N // RB2,),
    in_specs=[_p2_spec(0), _p2_spec(1),
              pl.BlockSpec((RB2, D), lambda i: (i, 0)),
              pl.BlockSpec((RB2, 1), lambda i: (i, 0)),
              _full_spec(1, D)],
    out_specs=pl.BlockSpec((RB2, D), lambda i: (i, 0)),
    out_shape=jax.ShapeDtypeStruct((N, D), jnp.float32),
)


@jax.jit
def kernel(x, edge_index, W1, b1, W2, b2):
    dst = edge_index[1].astype(jnp.int32)
    dst3 = dst.reshape(NW, NCH, K)
    dst1 = dst.reshape(NW, EPW)

    deg_parts = _sc_deg(dst1)

    ei2 = lax.optimization_barrier(edge_index)
    src1 = ei2[0].astype(jnp.int32).reshape(NW, EPW)

    xp = jnp.zeros((NPAD, D), jnp.float32).at[:N].set(x)
    zrows = jnp.zeros((STRIPE, D), jnp.float32)

    dv, hs1 = _tc_mmdvhs(xp, deg_parts.T, W1)

    p1 = _sc_scatter(hs1, src1, dst3, zrows)
    hs2 = _tc_fuse1(p1, p1, hs1, dv, b1.reshape(1, D), W2)
    p2 = _sc_scatter(hs2, src1, dst3, zrows)
    out = _tc_fuse2(p2, p2, hs2, dv, b2.reshape(1, D))
    return out


# speedup vs baseline: 1.0319x; 1.0319x over previous
import functools

import jax
import jax.numpy as jnp
from jax import lax
from jax.experimental import pallas as pl
from jax.experimental.pallas import tpu as pltpu
from jax.experimental.pallas import tpu_sc as plsc

N = 10000
E = 320000
D = 128

NCC = 2
NS = 16
NW = NCC * NS

NPAD = 10112
TRASH = 10000
STRIPE = NPAD // NS

K = 80
NCH = 125
NBUF = 2
EPW = K * NCH

RB = 1264


def _sc_deg_body(dst_hbm, deg_out, didx, deg):
    cid = lax.axis_index("c")
    sid = lax.axis_index("s")
    wid = cid * NS + sid
    pltpu.sync_copy(dst_hbm.at[wid], didx)

    def zero(i, _):
        deg[pl.ds(i * 16, 16)] = jnp.zeros((16,), jnp.float32)
        return _

    lax.fori_loop(0, NPAD // 16, zero, None)

    def acc(i, _):
        idx = didx[pl.ds(i * 16, 16)]
        plsc.addupdate_scatter(deg, [idx], jnp.ones((16,), jnp.float32))
        return _

    lax.fori_loop(0, EPW // 16, acc, None)
    pltpu.sync_copy(deg, deg_out.at[wid])


_sc_deg = pl.kernel(
    _sc_deg_body,
    out_type=jax.ShapeDtypeStruct((NW, NPAD), jnp.float32),
    mesh=plsc.VectorSubcoreMesh(core_axis_name="c", subcore_axis_name="s",
                                num_cores=NCC, num_subcores=NS),
    scratch_types=[
        pltpu.VMEM((EPW,), jnp.int32),
        pltpu.VMEM((NPAD,), jnp.float32),
    ],
    compiler_params=pltpu.CompilerParams(needs_layout_passes=False),
)


def _sc_scatter_body(hs, src1, dst3, zrows, out, sidx, didx, rows,
                     accum, gsem, ssem):
    cid = lax.axis_index("c")
    sid = lax.axis_index("s")
    wid = cid * NS + sid

    pltpu.sync_copy(src1.at[wid], sidx)
    pltpu.sync_copy(dst3.at[wid], didx)

    pltpu.sync_copy(zrows, accum.at[pl.ds(sid * STRIPE, STRIPE)])
    plsc.subcore_barrier()

    for b in range(NBUF):
        pltpu.async_copy(hs.at[sidx.at[pl.ds(b * K, K)]], rows.at[b],
                         gsem.at[b])

    def grp(g, _):
        for b in range(NBUF):
            j = g * NBUF + b
            pltpu.make_async_copy(hs.at[sidx.at[pl.ds(j * K, K)]],
                                  rows.at[b], gsem.at[b]).wait()
            pltpu.async_copy(rows.at[b], accum.at[didx.at[j]], ssem.at[b],
                             add=True)
        for b in range(NBUF):
            j = g * NBUF + b
            pltpu.make_async_copy(rows.at[b], accum.at[didx.at[j]],
                                  ssem.at[b]).wait()
            nj = j + NBUF

            @pl.when(nj < NCH)
            def _issue():
                pltpu.async_copy(hs.at[sidx.at[pl.ds(nj * K, K)]],
                                 rows.at[b], gsem.at[b])

        return _

    lax.fori_loop(0, NCH // NBUF, grp, None)
    for j in range(NCH - NCH % NBUF, NCH):
        b = j % NBUF
        pltpu.make_async_copy(hs.at[sidx.at[pl.ds(j * K, K)]],
                              rows.at[b], gsem.at[b]).wait()
        pltpu.async_copy(rows.at[b], accum.at[didx.at[j]], ssem.at[b],
                         add=True)
        pltpu.make_async_copy(rows.at[b], accum.at[didx.at[j]],
                              ssem.at[b]).wait()
    plsc.subcore_barrier()

    pltpu.sync_copy(accum.at[pl.ds(sid * STRIPE, STRIPE)],
                    out.at[cid, pl.ds(sid * STRIPE, STRIPE)])


_sc_scatter = pl.kernel(
    _sc_scatter_body,
    out_type=jax.ShapeDtypeStruct((NCC, NPAD, D), jnp.float32),
    mesh=plsc.VectorSubcoreMesh(core_axis_name="c", subcore_axis_name="s",
                                num_cores=NCC, num_subcores=NS),
    scratch_types=[
        pltpu.VMEM((EPW,), jnp.int32),
        pltpu.VMEM((NCH, K), jnp.int32),
        pltpu.VMEM((NBUF, K, D), jnp.float32),
        pltpu.VMEM_SHARED((NPAD, D), jnp.float32),
        pltpu.SemaphoreType.DMA((NBUF,)),
        pltpu.SemaphoreType.DMA((NBUF,)),
    ],
    compiler_params=pltpu.CompilerParams(needs_layout_passes=False),
)


def _tc_mmdvhs_body(x_ref, degT_ref, w_ref, dv_ref, hs_ref):
    s = jnp.sum(degT_ref[...], axis=1, keepdims=True) + 1.0
    dv = lax.rsqrt(s)
    dv_ref[...] = dv
    hs_ref[...] = jnp.dot(x_ref[...], w_ref[...],
                          preferred_element_type=jnp.float32) * dv


def _tc_fuse1_body(a_ref, b_ref, hs_ref, dv_ref, b1_ref, w2_ref, o_ref):
    z = (a_ref[0] + b_ref[0] + hs_ref[...]) * dv_ref[...] + b1_ref[...]
    z = jnp.maximum(z, 0.0)
    o_ref[...] = jnp.dot(z, w2_ref[...],
                         preferred_element_type=jnp.float32) * dv_ref[...]


def _tc_fuse2_body(a_ref, b_ref, hs_ref, dv_ref, b2_ref, o_ref):
    o_ref[...] = (a_ref[0] + b_ref[0] + hs_ref[...]) * dv_ref[...] + b2_ref[...]


_GRID = NPAD // RB


def _row_spec(w):
    return pl.BlockSpec((RB, w), lambda i: (i, 0))


def _full_spec(h, w):
    return pl.BlockSpec((h, w), lambda i: (0, 0))


def _part_spec(c):
    return pl.BlockSpec((1, RB, D), lambda i, c=c: (c, i, 0))


_tc_mmdvhs = pl.pallas_call(
    _tc_mmdvhs_body,
    grid=(_GRID,),
    in_specs=[_row_spec(D), _row_spec(NW), _full_spec(D, D)],
    out_specs=(_row_spec(1), _row_spec(D)),
    out_shape=(jax.ShapeDtypeStruct((NPAD, 1), jnp.float32),
               jax.ShapeDtypeStruct((NPAD, D), jnp.float32)),
)

_tc_fuse1 = pl.pallas_call(
    _tc_fuse1_body,
    grid=(_GRID,),
    in_specs=[_part_spec(0), _part_spec(1), _row_spec(D), _row_spec(1),
              _full_spec(1, D), _full_spec(D, D)],
    out_specs=_row_spec(D),
    out_shape=jax.ShapeDtypeStruct((NPAD, D), jnp.float32),
)

RB2 = 2000


def _p2_spec(c):
    return pl.BlockSpec((1, RB2, D), lambda i, c=c: (c, i, 0))


_tc_fuse2 = pl.pallas_call(
    _tc_fuse2_body,
    grid=(N // RB2,),
    in_specs=[_p2_spec(0), _p2_spec(1),
              pl.BlockSpec((RB2, D), lambda i: (i, 0)),
              pl.BlockSpec((RB2, 1), lambda i: (i, 0)),
              _full_spec(1, D)],
    out_specs=pl.BlockSpec((RB2, D), lambda i: (i, 0)),
    out_shape=jax.ShapeDtypeStruct((N, D), jnp.float32),
)


@jax.jit
def kernel(x, edge_index, W1, b1, W2, b2):
    ei = edge_index.astype(jnp.int32)
    dst3 = ei[1].reshape(NW, NCH, K)
    dst1 = ei[1].reshape(NW, EPW)

    deg_parts = _sc_deg(dst1)

    src1 = ei[0].reshape(NW, EPW)

    xp = jnp.zeros((NPAD, D), jnp.float32).at[:N].set(x)
    zrows = jnp.zeros((STRIPE, D), jnp.float32)

    dv, hs1 = _tc_mmdvhs(xp, deg_parts.T, W1)

    p1 = _sc_scatter(hs1, src1, dst3, zrows)
    hs2 = _tc_fuse1(p1, p1, hs1, dv, b1.reshape(1, D), W2)
    p2 = _sc_scatter(hs2, src1, dst3, zrows)
    out = _tc_fuse2(p2, p2, hs2, dv, b2.reshape(1, D))
    return out
